# Initial kernel scaffold; baseline (speedup 1.0000x reference)
import jax
import jax.numpy as jnp
from jax.experimental import pallas as pl


def kernel(node_features, cond_tokens, node2graph, frac_coords, lattices, edges, edge2graph, Wq, bq, Wk, bk, Wv, bv, Wo, bo, sp1W, sp1b, sp2W, sp2b, e1W, e1b, e2W, e2b, n1W, n1b, n2W, n2b):
    raise NotImplementedError("write your pallas kernel here")


# speedup vs baseline: 3.7680x; 3.7680x over previous
import dataclasses
import functools

import jax
import jax.numpy as jnp
from jax import lax
from jax.experimental import pallas as pl
from jax.experimental.pallas import tpu as pltpu
from jax.experimental.pallas import tpu_sc as plsc

HID = 128
NT = 8
NF = 4
NB = 64
NNODE = 10000
NPAD = 10240
NEDGE = 320000
BN = 2000
BE = 2000
SC_C = 80
NCORES = 2
NSUB = 16
NWORK = NCORES * NSUB
LANES = 16


def _silu(x):
    return x * jax.nn.sigmoid(x)


def _prep_body(nf_ref, n2g_ref, fc_ref, cond_ref, lat9_ref,
               wq_ref, bq_ref, wk_ref, bk_ref, wv_ref, bv_ref, wo_ref, bo_ref,
               sp1w_ref, sp1b_ref, sp2w_ref, sp2b_ref,
               whi_ref, whj_ref, wlat_ref, wfd_ref, e1b_ref,
               nf1_ref, t1_ref, t2_ref, latc_ref):
    nf = nf_ref[...]
    cond = cond_ref[...]
    kall = jnp.dot(cond, wk_ref[...], preferred_element_type=jnp.float32) + bk_ref[...]
    vall = jnp.dot(cond, wv_ref[...], preferred_element_type=jnp.float32) + bv_ref[...]
    q = jnp.dot(nf, wq_ref[...], preferred_element_type=jnp.float32) + bq_ref[...]
    scores = lax.dot_general(q, kall, (((1,), (1,)), ((), ())),
                             preferred_element_type=jnp.float32)
    scores = scores * (1.0 / jnp.sqrt(jnp.float32(HID)))

    fc = fc_ref[...]
    fr = jnp.exp2(lax.broadcasted_iota(jnp.int32, (1, NF), 1)
                  .astype(jnp.float32)) * jnp.pi
    ph = jnp.concatenate([fc[:, j:j + 1] * fr for j in range(3)], axis=1)
    sp = jnp.concatenate([jnp.sin(ph), jnp.cos(ph)], axis=1)
    sb = jnp.dot(_silu(jnp.dot(sp, sp1w_ref[...],
                               preferred_element_type=jnp.float32) + sp1b_ref[...]),
                 sp2w_ref[...], preferred_element_type=jnp.float32) + sp2b_ref[...]

    tr = lax.broadcasted_iota(jnp.int32, (NT, NB * NT), 0)
    tc = lax.broadcasted_iota(jnp.int32, (NT, NB * NT), 1)
    tile8 = (tc % NT == tr).astype(jnp.float32)
    bias512 = jnp.dot(sb, tile8, preferred_element_type=jnp.float32)

    colg = lax.broadcasted_iota(jnp.int32, (scores.shape[0], NB * NT), 1) // NT
    mask = n2g_ref[...] == colg
    logits = jnp.where(mask, scores + bias512, -1e30)
    mx = jnp.max(logits, axis=1, keepdims=True)
    ex = jnp.exp(logits - mx)
    p = ex / jnp.sum(ex, axis=1, keepdims=True)
    attn_out = jnp.dot(p, vall, preferred_element_type=jnp.float32)
    nf1 = nf + jnp.dot(attn_out, wo_ref[...],
                       preferred_element_type=jnp.float32) + bo_ref[...]
    nf1_ref[...] = nf1

    fcw = jnp.dot(fc, wfd_ref[...], preferred_element_type=jnp.float32)
    t1_ref[...] = jnp.dot(nf1, whi_ref[...],
                          preferred_element_type=jnp.float32) - fcw
    t2_ref[...] = jnp.dot(nf1, whj_ref[...],
                          preferred_element_type=jnp.float32) + fcw

    lat9 = lat9_ref[...]
    ipcols = []
    for i in range(3):
        for k in range(3):
            s = (lat9[:, 3 * i:3 * i + 1] * lat9[:, 3 * k:3 * k + 1]
                 + lat9[:, 3 * i + 1:3 * i + 2] * lat9[:, 3 * k + 1:3 * k + 2]
                 + lat9[:, 3 * i + 2:3 * i + 3] * lat9[:, 3 * k + 2:3 * k + 3])
            ipcols.append(s)
    ips = jnp.concatenate(ipcols, axis=1)
    latc_ref[...] = jnp.dot(ips, wlat_ref[...],
                            preferred_element_type=jnp.float32) + e1b_ref[...]


def _prep_call(nf, n2g, fc, cond_flat, lat9, Wq, bq, Wk, bk, Wv, bv, Wo, bo,
               sp1W, sp1b, sp2W, sp2b, W_hi, W_hj, W_lat, W_fd, e1b):
    grid = (NNODE // BN,)
    full = lambda shape: pl.BlockSpec(shape, lambda i: (0, 0))
    blk = lambda w: pl.BlockSpec((BN, w), lambda i: (i, 0))
    return pl.pallas_call(
        _prep_body,
        grid=grid,
        in_specs=[
            blk(HID), blk(1), blk(3), full((NB * NT, HID)), full((NB, 9)),
            full((HID, HID)), full((1, HID)), full((HID, HID)), full((1, HID)),
            full((HID, HID)), full((1, HID)), full((HID, HID)), full((1, HID)),
            full((6 * NF, HID)), full((1, HID)), full((HID, NT)), full((1, NT)),
            full((HID, HID)), full((HID, HID)), full((9, HID)), full((3, HID)),
            full((1, HID)),
        ],
        out_specs=[blk(HID), blk(HID), blk(HID), full((NB, HID))],
        out_shape=[
            jax.ShapeDtypeStruct((NNODE, HID), jnp.float32),
            jax.ShapeDtypeStruct((NNODE, HID), jnp.float32),
            jax.ShapeDtypeStruct((NNODE, HID), jnp.float32),
            jax.ShapeDtypeStruct((NB, HID), jnp.float32),
        ],
    )(nf, n2g, fc, cond_flat, lat9, Wq, bq, Wk, bk, Wv, bv, Wo, bo,
      sp1W, sp1b, sp2W, sp2b, W_hi, W_hj, W_lat, W_fd, e1b)


def _sc_compiler_params():
    cp = pltpu.CompilerParams()
    if "needs_layout_passes" in pltpu.CompilerParams.__dataclass_fields__:
        cp = dataclasses.replace(cp, needs_layout_passes=False)
    return cp


def _sc_gather(T1, T2, src, dst, fc0, fc1, fc2):
    mesh = plsc.VectorSubcoreMesh(core_axis_name="c", subcore_axis_name="s")
    ew = NEDGE // NWORK

    @functools.partial(
        pl.kernel,
        compiler_params=_sc_compiler_params(),
        out_type=(jax.ShapeDtypeStruct((NEDGE, HID), jnp.float32),
                  jax.ShapeDtypeStruct((NEDGE, HID), jnp.float32),
                  jax.ShapeDtypeStruct((NEDGE,), jnp.int32)),
        mesh=mesh,
        scratch_types=[pltpu.VMEM((SC_C,), jnp.int32),
                       pltpu.VMEM((SC_C, HID), jnp.float32),
                       pltpu.VMEM((SC_C,), jnp.int32),
                       pltpu.VMEM((SC_C, HID), jnp.float32),
                       pltpu.VMEM((SC_C,), jnp.int32),
                       pltpu.VMEM((NNODE,), jnp.float32),
                       pltpu.VMEM((NNODE,), jnp.float32),
                       pltpu.VMEM((NNODE,), jnp.float32),
                       pltpu.SemaphoreType.DMA,
                       pltpu.SemaphoreType.DMA],
    )
    def gk(t1, t2, s_idx, d_idx, f0, f1, f2, g1, g2, codes,
           idx1_v, rows1_v, idx2_v, rows2_v, m_v, f0_v, f1_v, f2_v,
           sem1, sem2):
        wid = lax.axis_index("s") * NCORES + lax.axis_index("c")
        base0 = wid * ew
        pltpu.sync_copy(f0, f0_v)
        pltpu.sync_copy(f1, f1_v)
        pltpu.sync_copy(f2, f2_v)
        fvs = (f0_v, f1_v, f2_v)

        @pl.loop(0, ew, step=SC_C)
        def _(off):
            base = base0 + off
            pltpu.sync_copy(s_idx.at[pl.ds(base, SC_C)], idx1_v)
            cp1 = pltpu.async_copy(t1.at[idx1_v], rows1_v, sem1)
            pltpu.sync_copy(d_idx.at[pl.ds(base, SC_C)], idx2_v)
            cp2 = pltpu.async_copy(t2.at[idx2_v], rows2_v, sem2)
            for g in range(SC_C // LANES):
                si = idx1_v[pl.ds(g * LANES, LANES)]
                di = idx2_v[pl.ds(g * LANES, LANES)]
                m = jnp.zeros((LANES,), jnp.int32)
                for j, fv in enumerate(fvs):
                    fs = plsc.load_gather(fv, [si])
                    fdv = plsc.load_gather(fv, [di])
                    m = m + jnp.where(fdv < fs,
                                      jnp.int32(1 << j), jnp.int32(0))
                m_v[pl.ds(g * LANES, LANES)] = m
            cp1.wait()
            pltpu.sync_copy(rows1_v, g1.at[pl.ds(base, SC_C)])
            cp2.wait()
            pltpu.sync_copy(rows2_v, g2.at[pl.ds(base, SC_C)])
            pltpu.sync_copy(m_v, codes.at[pl.ds(base, SC_C)])

    return gk(T1, T2, src, dst, fc0, fc1, fc2)


def _edge_body(g1_ref, g2_ref, e2g_ref, code_ref, latc_ref, wfd_ref,
               w2_ref, b2_ref, ef_ref):
    g1 = g1_ref[...]
    g2 = g2_ref[...]
    n = g1.shape[0]
    colb = lax.broadcasted_iota(jnp.int32, (n, NB), 1)
    onehot = (e2g_ref[...] == colb).astype(jnp.float32)
    pre = g1 + g2 + jnp.dot(onehot, latc_ref[...],
                            preferred_element_type=jnp.float32)
    wfd = wfd_ref[...]
    rows = []
    for m in range(8):
        r = jnp.zeros((1, HID), jnp.float32)
        for j in range(3):
            if m & (1 << j):
                r = r + wfd[j:j + 1, :]
        rows.append(r)
    corr8 = jnp.concatenate(rows, axis=0)
    col8 = lax.broadcasted_iota(jnp.int32, (n, 8), 1)
    oh8 = (code_ref[...] == col8).astype(jnp.float32)
    pre = pre + jnp.dot(oh8, corr8, preferred_element_type=jnp.float32)
    h = _silu(pre)
    ef_ref[...] = _silu(jnp.dot(h, w2_ref[...],
                                preferred_element_type=jnp.float32)
                        + b2_ref[...])


def _edge_call(G1, G2, e2g, codes2, latc, W_fd, e2W, e2b):
    grid = (NEDGE // BE,)
    full = lambda shape: pl.BlockSpec(shape, lambda i: (0, 0))
    blk = lambda w: pl.BlockSpec((BE, w), lambda i: (i, 0))
    return pl.pallas_call(
        _edge_body,
        grid=grid,
        in_specs=[blk(HID), blk(HID), blk(1), blk(1), full((NB, HID)),
                  full((3, HID)), full((HID, HID)), full((1, HID))],
        out_specs=blk(HID),
        out_shape=jax.ShapeDtypeStruct((NEDGE, HID), jnp.float32),
    )(G1, G2, e2g, codes2, latc, W_fd, e2W, e2b)


def _sc_scatter(ef, src, zeros_init, ones_rows):
    mesh = plsc.VectorSubcoreMesh(core_axis_name="c", subcore_axis_name="s")
    ec = NEDGE // NCORES
    ew = ec // NSUB
    rows_per = NPAD // NSUB

    @functools.partial(
        pl.kernel,
        out_type=(jax.ShapeDtypeStruct((NCORES * NPAD, HID), jnp.float32),
                  jax.ShapeDtypeStruct((NCORES * NPAD, HID), jnp.float32)),
        mesh=mesh,
        scratch_types=[pltpu.VMEM((SC_C,), jnp.int32),
                       pltpu.VMEM((SC_C, HID), jnp.float32),
                       pltpu.VMEM_SHARED((NPAD, HID), jnp.float32)],
    )
    def sk(ef_h, src_h, z_h, ones_h, out_h, cnt_h, idx_v, rows_v, acc):
        c = lax.axis_index("c")
        s = lax.axis_index("s")
        base0 = c * ec + s * ew

        @pl.when(s == 0)
        def _():
            pltpu.sync_copy(z_h, acc)

        plsc.subcore_barrier()

        @pl.loop(0, ew, step=SC_C)
        def _(off):
            base = base0 + off
            pltpu.sync_copy(src_h.at[pl.ds(base, SC_C)], idx_v)
            pltpu.sync_copy(ef_h.at[pl.ds(base, SC_C)], rows_v)
            pltpu.sync_copy(rows_v, acc.at[idx_v], add=True)

        plsc.subcore_barrier()
        pltpu.sync_copy(acc.at[pl.ds(s * rows_per, rows_per)],
                        out_h.at[pl.ds(c * NPAD + s * rows_per, rows_per)])
        plsc.subcore_barrier()

        @pl.when(s == 0)
        def _():
            pltpu.sync_copy(z_h, acc)

        plsc.subcore_barrier()
        pltpu.sync_copy(ones_h, rows_v)

        @pl.loop(0, ew, step=SC_C)
        def _(off):
            base = base0 + off
            pltpu.sync_copy(src_h.at[pl.ds(base, SC_C)], idx_v)
            pltpu.sync_copy(rows_v, acc.at[idx_v], add=True)

        plsc.subcore_barrier()
        pltpu.sync_copy(acc.at[pl.ds(s * rows_per, rows_per)],
                        cnt_h.at[pl.ds(c * NPAD + s * rows_per, rows_per)])

    return sk(ef, src, zeros_init, ones_rows)


def _node_body(nf0_ref, nf1_ref, a0_ref, a1_ref, c0_ref, c1_ref,
               w1_ref, b1_ref, w2_ref, b2_ref, out_ref):
    s = a0_ref[...] + a1_ref[...]
    cnt = jnp.maximum(c0_ref[:, :1] + c1_ref[:, :1], 1.0)
    agg = s / cnt
    nf1 = nf1_ref[...]
    nin = jnp.concatenate([nf1, agg], axis=1)
    h = _silu(jnp.dot(nin, w1_ref[...], preferred_element_type=jnp.float32)
              + b1_ref[...])
    nout = _silu(jnp.dot(h, w2_ref[...], preferred_element_type=jnp.float32)
                 + b2_ref[...])
    out_ref[...] = nf0_ref[...] + nout


def _node_call(nf0, nf1, acc0, acc1, cnt0, cnt1, n1W, n1b, n2W, n2b):
    grid = (NNODE // BN,)
    full = lambda shape: pl.BlockSpec(shape, lambda i: (0, 0))
    blk = lambda w: pl.BlockSpec((BN, w), lambda i: (i, 0))
    return pl.pallas_call(
        _node_body,
        grid=grid,
        in_specs=[blk(HID), blk(HID), blk(HID), blk(HID), blk(HID), blk(HID),
                  full((2 * HID, HID)), full((1, HID)),
                  full((HID, HID)), full((1, HID))],
        out_specs=blk(HID),
        out_shape=jax.ShapeDtypeStruct((NNODE, HID), jnp.float32),
    )(nf0, nf1, acc0, acc1, cnt0, cnt1, n1W, n1b, n2W, n2b)


def kernel(node_features, cond_tokens, node2graph, frac_coords, lattices,
           edges, edge2graph, Wq, bq, Wk, bk, Wv, bv, Wo, bo,
           sp1W, sp1b, sp2W, sp2b, e1W, e1b, e2W, e2b, n1W, n1b, n2W, n2b):
    n2g = node2graph.astype(jnp.int32).reshape(NNODE, 1)
    e2g = edge2graph.astype(jnp.int32).reshape(NEDGE, 1)
    src = edges[0].astype(jnp.int32)
    dst = edges[1].astype(jnp.int32)
    cond_flat = cond_tokens.reshape(NB * NT, HID)
    lat9 = lattices.reshape(NB, 9)
    row = lambda b: b.reshape(1, -1)
    W_fd = e1W[2 * HID + 9:]

    nf1, T1, T2, latc = _prep_call(
        node_features, n2g, frac_coords, cond_flat, lat9,
        Wq, row(bq), Wk, row(bk), Wv, row(bv), Wo, row(bo),
        sp1W, row(sp1b), sp2W, row(sp2b),
        e1W[:HID], e1W[HID:2 * HID], e1W[2 * HID:2 * HID + 9], W_fd,
        row(e1b))

    G1, G2, codes = _sc_gather(T1, T2, src, dst,
                               frac_coords[:, 0], frac_coords[:, 1],
                               frac_coords[:, 2])

    EF = _edge_call(G1, G2, e2g, codes.reshape(NEDGE, 1), latc, W_fd,
                    e2W, row(e2b))

    zeros_init = jnp.zeros((NPAD, HID), jnp.float32)
    ones_rows = jnp.ones((SC_C, HID), jnp.float32)
    ACC, CNT = _sc_scatter(EF, src, zeros_init, ones_rows)

    return _node_call(node_features, nf1, ACC[:NNODE], ACC[NPAD:NPAD + NNODE],
                      CNT[:NNODE], CNT[NPAD:NPAD + NNODE],
                      n1W, row(n1b), n2W, row(n2b))


# speedup vs baseline: 4.8009x; 1.2741x over previous
import dataclasses
import functools

import jax
import jax.numpy as jnp
from jax import lax
from jax.experimental import pallas as pl
from jax.experimental.pallas import tpu as pltpu
from jax.experimental.pallas import tpu_sc as plsc

HID = 128
NT = 8
NF = 4
NB = 64
NNODE = 10000
NPAD = 10240
NEDGE = 320000
BN = 2000
BE = 2000
SC_C = 80
NCORES = 2
NSUB = 16
NWORK = NCORES * NSUB
LANES = 16


def _silu(x):
    return x * jax.nn.sigmoid(x)


def _prep_body(nf_ref, n2g_ref, fc_ref, cond_ref, lat9_ref,
               wq_ref, bq_ref, wk_ref, bk_ref, wv_ref, bv_ref, wo_ref, bo_ref,
               sp1w_ref, sp1b_ref, sp2w_ref, sp2b_ref,
               whi_ref, whj_ref, wlat_ref, wfd_ref, e1b_ref,
               nf1_ref, t1_ref, t2_ref, latc_ref):
    nf = nf_ref[...]
    cond = cond_ref[...]
    kall = jnp.dot(cond, wk_ref[...], preferred_element_type=jnp.float32) + bk_ref[...]
    vall = jnp.dot(cond, wv_ref[...], preferred_element_type=jnp.float32) + bv_ref[...]
    q = jnp.dot(nf, wq_ref[...], preferred_element_type=jnp.float32) + bq_ref[...]
    scores = lax.dot_general(q, kall, (((1,), (1,)), ((), ())),
                             preferred_element_type=jnp.float32)
    scores = scores * (1.0 / jnp.sqrt(jnp.float32(HID)))

    fc = fc_ref[...]
    fr = jnp.exp2(lax.broadcasted_iota(jnp.int32, (1, NF), 1)
                  .astype(jnp.float32)) * jnp.pi
    ph = jnp.concatenate([fc[:, j:j + 1] * fr for j in range(3)], axis=1)
    sp = jnp.concatenate([jnp.sin(ph), jnp.cos(ph)], axis=1)
    sb = jnp.dot(_silu(jnp.dot(sp, sp1w_ref[...],
                               preferred_element_type=jnp.float32) + sp1b_ref[...]),
                 sp2w_ref[...], preferred_element_type=jnp.float32) + sp2b_ref[...]

    tr = lax.broadcasted_iota(jnp.int32, (NT, NB * NT), 0)
    tc = lax.broadcasted_iota(jnp.int32, (NT, NB * NT), 1)
    tile8 = (tc % NT == tr).astype(jnp.float32)
    bias512 = jnp.dot(sb, tile8, preferred_element_type=jnp.float32)

    colg = lax.broadcasted_iota(jnp.int32, (scores.shape[0], NB * NT), 1) // NT
    mask = n2g_ref[...] == colg
    logits = jnp.where(mask, scores + bias512, -1e30)
    mx = jnp.max(logits, axis=1, keepdims=True)
    ex = jnp.exp(logits - mx)
    p = ex / jnp.sum(ex, axis=1, keepdims=True)
    attn_out = jnp.dot(p, vall, preferred_element_type=jnp.float32)
    nf1 = nf + jnp.dot(attn_out, wo_ref[...],
                       preferred_element_type=jnp.float32) + bo_ref[...]
    nf1_ref[...] = nf1

    fcw = jnp.dot(fc, wfd_ref[...], preferred_element_type=jnp.float32)
    t1_ref[...] = jnp.dot(nf1, whi_ref[...],
                          preferred_element_type=jnp.float32) - fcw
    t2_ref[...] = jnp.dot(nf1, whj_ref[...],
                          preferred_element_type=jnp.float32) + fcw

    lat9 = lat9_ref[...]
    ipcols = []
    for i in range(3):
        for k in range(3):
            s = (lat9[:, 3 * i:3 * i + 1] * lat9[:, 3 * k:3 * k + 1]
                 + lat9[:, 3 * i + 1:3 * i + 2] * lat9[:, 3 * k + 1:3 * k + 2]
                 + lat9[:, 3 * i + 2:3 * i + 3] * lat9[:, 3 * k + 2:3 * k + 3])
            ipcols.append(s)
    ips = jnp.concatenate(ipcols, axis=1)
    latc_ref[...] = jnp.dot(ips, wlat_ref[...],
                            preferred_element_type=jnp.float32) + e1b_ref[...]


def _prep_call(nf, n2g, fc, cond_flat, lat9, Wq, bq, Wk, bk, Wv, bv, Wo, bo,
               sp1W, sp1b, sp2W, sp2b, W_hi, W_hj, W_lat, W_fd, e1b):
    grid = (NNODE // BN,)
    full = lambda shape: pl.BlockSpec(shape, lambda i: (0, 0))
    blk = lambda w: pl.BlockSpec((BN, w), lambda i: (i, 0))
    return pl.pallas_call(
        _prep_body,
        grid=grid,
        in_specs=[
            blk(HID), blk(1), blk(3), full((NB * NT, HID)), full((NB, 9)),
            full((HID, HID)), full((1, HID)), full((HID, HID)), full((1, HID)),
            full((HID, HID)), full((1, HID)), full((HID, HID)), full((1, HID)),
            full((6 * NF, HID)), full((1, HID)), full((HID, NT)), full((1, NT)),
            full((HID, HID)), full((HID, HID)), full((9, HID)), full((3, HID)),
            full((1, HID)),
        ],
        out_specs=[blk(HID), blk(HID), blk(HID), full((NB, HID))],
        out_shape=[
            jax.ShapeDtypeStruct((NNODE, HID), jnp.float32),
            jax.ShapeDtypeStruct((NNODE, HID), jnp.float32),
            jax.ShapeDtypeStruct((NNODE, HID), jnp.float32),
            jax.ShapeDtypeStruct((NB, HID), jnp.float32),
        ],
    )(nf, n2g, fc, cond_flat, lat9, Wq, bq, Wk, bk, Wv, bv, Wo, bo,
      sp1W, sp1b, sp2W, sp2b, W_hi, W_hj, W_lat, W_fd, e1b)


def _sc_compiler_params():
    cp = pltpu.CompilerParams()
    if "needs_layout_passes" in pltpu.CompilerParams.__dataclass_fields__:
        cp = dataclasses.replace(cp, needs_layout_passes=False)
    return cp


def _sc_gather(T1, T2, src2d, dst2d, fc0, fc1, fc2):
    mesh = plsc.VectorSubcoreMesh(core_axis_name="c", subcore_axis_name="s")

    @functools.partial(
        pl.kernel,
        compiler_params=_sc_compiler_params(),
        out_type=(jax.ShapeDtypeStruct((NEDGE, HID), jnp.float32),
                  jax.ShapeDtypeStruct((NEDGE, HID), jnp.float32),
                  jax.ShapeDtypeStruct((NEDGE // SC_C, 1, SC_C), jnp.int32)),
        mesh=mesh,
        scratch_types=[pltpu.VMEM((NNODE,), jnp.float32),
                       pltpu.VMEM((NNODE,), jnp.float32),
                       pltpu.VMEM((NNODE,), jnp.float32),
                       pltpu.SemaphoreType.DMA,
                       pltpu.SemaphoreType.DMA],
    )
    def gk(t1, t2, s_idx, d_idx, f0, f1, f2, g1, g2, codes,
           f0_v, f1_v, f2_v, sem1, sem2):
        pltpu.sync_copy(f0, f0_v)
        pltpu.sync_copy(f1, f1_v)
        pltpu.sync_copy(f2, f2_v)
        fvs = (f0_v, f1_v, f2_v)

        def body(sidx_v, didx_v, g1_v, g2_v, m_v):
            cp1 = pltpu.async_copy(t1.at[sidx_v.at[0, 0]], g1_v, sem1)
            cp2 = pltpu.async_copy(t2.at[didx_v.at[0, 0]], g2_v, sem2)
            for g in range(SC_C // LANES):
                si = sidx_v[0, 0, pl.ds(g * LANES, LANES)]
                di = didx_v[0, 0, pl.ds(g * LANES, LANES)]
                m = jnp.zeros((LANES,), jnp.int32)
                for j, fv in enumerate(fvs):
                    fs = plsc.load_gather(fv, [si])
                    fdv = plsc.load_gather(fv, [di])
                    m = m + jnp.where(fdv < fs,
                                      jnp.int32(1 << j), jnp.int32(0))
                m_v[0, 0, pl.ds(g * LANES, LANES)] = m
            cp1.wait()
            cp2.wait()

        pltpu.emit_pipeline(
            body,
            grid=(NEDGE // SC_C,),
            in_specs=[pl.BlockSpec((1, 1, SC_C), lambda i: (i, 0, 0)),
                      pl.BlockSpec((1, 1, SC_C), lambda i: (i, 0, 0))],
            out_specs=[pl.BlockSpec((SC_C, HID), lambda i: (i, 0)),
                       pl.BlockSpec((SC_C, HID), lambda i: (i, 0)),
                       pl.BlockSpec((1, 1, SC_C), lambda i: (i, 0, 0))],
            core_axis_name=("c", "s"),
            dimension_semantics=(pltpu.PARALLEL,),
        )(s_idx, d_idx, g1, g2, codes)

    return gk(T1, T2, src2d, dst2d, fc0, fc1, fc2)


def _edge_body(g1_ref, g2_ref, e2g_ref, code_ref, latc_ref, wfd_ref,
               w2_ref, b2_ref, ef_ref):
    g1 = g1_ref[...]
    g2 = g2_ref[...]
    n = g1.shape[0]
    colb = lax.broadcasted_iota(jnp.int32, (n, NB), 1)
    onehot = (e2g_ref[...] == colb).astype(jnp.float32)
    pre = g1 + g2 + jnp.dot(onehot, latc_ref[...],
                            preferred_element_type=jnp.float32)
    wfd = wfd_ref[...]
    rows = []
    for m in range(8):
        r = jnp.zeros((1, HID), jnp.float32)
        for j in range(3):
            if m & (1 << j):
                r = r + wfd[j:j + 1, :]
        rows.append(r)
    corr8 = jnp.concatenate(rows, axis=0)
    col8 = lax.broadcasted_iota(jnp.int32, (n, 8), 1)
    oh8 = (code_ref[...] == col8).astype(jnp.float32)
    pre = pre + jnp.dot(oh8, corr8, preferred_element_type=jnp.float32)
    h = _silu(pre)
    ef_ref[...] = _silu(jnp.dot(h, w2_ref[...],
                                preferred_element_type=jnp.float32)
                        + b2_ref[...])


def _edge_call(G1, G2, e2g, codes2, latc, W_fd, e2W, e2b):
    grid = (NEDGE // BE,)
    full = lambda shape: pl.BlockSpec(shape, lambda i: (0, 0))
    blk = lambda w: pl.BlockSpec((BE, w), lambda i: (i, 0))
    return pl.pallas_call(
        _edge_body,
        grid=grid,
        in_specs=[blk(HID), blk(HID), blk(1), blk(1), full((NB, HID)),
                  full((3, HID)), full((HID, HID)), full((1, HID))],
        out_specs=blk(HID),
        out_shape=jax.ShapeDtypeStruct((NEDGE, HID), jnp.float32),
    )(G1, G2, e2g, codes2, latc, W_fd, e2W, e2b)


def _sc_scatter(ef, src2d, zeros_init, ones_rows):
    mesh = plsc.VectorSubcoreMesh(core_axis_name="c", subcore_axis_name="s")
    rows_per = NPAD // NSUB

    @functools.partial(
        pl.kernel,
        out_type=(jax.ShapeDtypeStruct((NCORES * NPAD, HID), jnp.float32),
                  jax.ShapeDtypeStruct((NCORES * NPAD, HID), jnp.float32)),
        mesh=mesh,
        scratch_types=[pltpu.VMEM((SC_C, HID), jnp.float32),
                       pltpu.VMEM_SHARED((NPAD, HID), jnp.float32)],
    )
    def sk(ef_h, src_h, z_h, ones_h, out_h, cnt_h, ones_v, acc):
        c = lax.axis_index("c")
        s = lax.axis_index("s")

        @pl.when(s == 0)
        def _():
            pltpu.sync_copy(z_h, acc)

        pltpu.sync_copy(ones_h, ones_v)
        plsc.subcore_barrier()

        def body1(idx_v, rows_v):
            pltpu.sync_copy(rows_v, acc.at[idx_v.at[0, 0]], add=True)

        pltpu.emit_pipeline(
            body1,
            grid=(NEDGE // SC_C,),
            in_specs=[pl.BlockSpec((1, 1, SC_C), lambda i: (i, 0, 0)),
                      pl.BlockSpec((SC_C, HID), lambda i: (i, 0))],
            out_specs=[],
            core_axis_name=("c", "s"),
            dimension_semantics=(pltpu.PARALLEL,),
        )(src_h, ef_h)

        plsc.subcore_barrier()
        pltpu.sync_copy(acc.at[pl.ds(s * rows_per, rows_per)],
                        out_h.at[pl.ds(c * NPAD + s * rows_per, rows_per)])
        plsc.subcore_barrier()

        @pl.when(s == 0)
        def _():
            pltpu.sync_copy(z_h, acc)

        plsc.subcore_barrier()

        def body2(idx_v):
            pltpu.sync_copy(ones_v, acc.at[idx_v.at[0, 0]], add=True)

        pltpu.emit_pipeline(
            body2,
            grid=(NEDGE // SC_C,),
            in_specs=[pl.BlockSpec((1, 1, SC_C), lambda i: (i, 0, 0))],
            out_specs=[],
            core_axis_name=("c", "s"),
            dimension_semantics=(pltpu.PARALLEL,),
        )(src_h)

        plsc.subcore_barrier()
        pltpu.sync_copy(acc.at[pl.ds(s * rows_per, rows_per)],
                        cnt_h.at[pl.ds(c * NPAD + s * rows_per, rows_per)])

    return sk(ef, src2d, zeros_init, ones_rows)


def _node_body(nf0_ref, nf1_ref, a0_ref, a1_ref, c0_ref, c1_ref,
               w1_ref, b1_ref, w2_ref, b2_ref, out_ref):
    s = a0_ref[...] + a1_ref[...]
    cnt = jnp.maximum(c0_ref[:, :1] + c1_ref[:, :1], 1.0)
    agg = s / cnt
    nf1 = nf1_ref[...]
    nin = jnp.concatenate([nf1, agg], axis=1)
    h = _silu(jnp.dot(nin, w1_ref[...], preferred_element_type=jnp.float32)
              + b1_ref[...])
    nout = _silu(jnp.dot(h, w2_ref[...], preferred_element_type=jnp.float32)
                 + b2_ref[...])
    out_ref[...] = nf0_ref[...] + nout


def _node_call(nf0, nf1, acc0, acc1, cnt0, cnt1, n1W, n1b, n2W, n2b):
    grid = (NNODE // BN,)
    full = lambda shape: pl.BlockSpec(shape, lambda i: (0, 0))
    blk = lambda w: pl.BlockSpec((BN, w), lambda i: (i, 0))
    return pl.pallas_call(
        _node_body,
        grid=grid,
        in_specs=[blk(HID), blk(HID), blk(HID), blk(HID), blk(HID), blk(HID),
                  full((2 * HID, HID)), full((1, HID)),
                  full((HID, HID)), full((1, HID))],
        out_specs=blk(HID),
        out_shape=jax.ShapeDtypeStruct((NNODE, HID), jnp.float32),
    )(nf0, nf1, acc0, acc1, cnt0, cnt1, n1W, n1b, n2W, n2b)


def kernel(node_features, cond_tokens, node2graph, frac_coords, lattices,
           edges, edge2graph, Wq, bq, Wk, bk, Wv, bv, Wo, bo,
           sp1W, sp1b, sp2W, sp2b, e1W, e1b, e2W, e2b, n1W, n1b, n2W, n2b):
    n2g = node2graph.astype(jnp.int32).reshape(NNODE, 1)
    e2g = edge2graph.astype(jnp.int32).reshape(NEDGE, 1)
    src = edges[0].astype(jnp.int32)
    dst = edges[1].astype(jnp.int32)
    cond_flat = cond_tokens.reshape(NB * NT, HID)
    lat9 = lattices.reshape(NB, 9)
    row = lambda b: b.reshape(1, -1)
    W_fd = e1W[2 * HID + 9:]

    nf1, T1, T2, latc = _prep_call(
        node_features, n2g, frac_coords, cond_flat, lat9,
        Wq, row(bq), Wk, row(bk), Wv, row(bv), Wo, row(bo),
        sp1W, row(sp1b), sp2W, row(sp2b),
        e1W[:HID], e1W[HID:2 * HID], e1W[2 * HID:2 * HID + 9], W_fd,
        row(e1b))

    src3 = src.reshape(NEDGE // SC_C, 1, SC_C)
    dst3 = dst.reshape(NEDGE // SC_C, 1, SC_C)
    G1, G2, codes = _sc_gather(T1, T2, src3, dst3,
                               frac_coords[:, 0], frac_coords[:, 1],
                               frac_coords[:, 2])

    EF = _edge_call(G1, G2, e2g, codes.reshape(NEDGE, 1), latc, W_fd,
                    e2W, row(e2b))

    zeros_init = jnp.zeros((NPAD, HID), jnp.float32)
    ones_rows = jnp.ones((SC_C, HID), jnp.float32)
    ACC, CNT = _sc_scatter(EF, src3, zeros_init, ones_rows)

    return _node_call(node_features, nf1, ACC[:NNODE], ACC[NPAD:NPAD + NNODE],
                      CNT[:NNODE], CNT[NPAD:NPAD + NNODE],
                      n1W, row(n1b), n2W, row(n2b))
